# baseline (speedup 1.0000x reference)
import functools

import jax
import jax.numpy as jnp
from jax import lax
from jax.experimental import pallas as pl
from jax.experimental.pallas import tpu as pltpu
from jax.experimental.pallas import tpu_sc as plsc

N = 4096
K = 32
R2 = 25.0
NB = 32
L = 16
NC, NS = 1, 16
NW = NC * NS
RPW = N // NW
NCHUNK = N // L
INF = float("inf")


def _lexless(ka, va, kb, vb):
    return (ka < kb) | ((ka == kb) & (va < vb))


def _merge16to32(Ck, Cv, Dk, Dv):
    rDk = jnp.flip(Dk, 0)
    rDv = jnp.flip(Dv, 0)
    lt = _lexless(Ck, Cv, rDk, rDv)
    P0k = jnp.where(lt, Ck, rDk)
    P0v = jnp.where(lt, Cv, rDv)
    P1k = jnp.where(lt, rDk, Ck)
    P1v = jnp.where(lt, rDv, Cv)
    S0k, S0v = plsc.sort_key_val(P0k, P0v)
    S1k, S1v = plsc.sort_key_val(P1k, P1v)
    return S0k, S0v, S1k, S1v


def _merge32keep32(T0k, T0v, T1k, T1v, S0k, S0v, S1k, S1v):
    rS0k = jnp.flip(S0k, 0)
    rS0v = jnp.flip(S0v, 0)
    rS1k = jnp.flip(S1k, 0)
    rS1v = jnp.flip(S1v, 0)
    ltA = _lexless(T0k, T0v, rS1k, rS1v)
    L0k = jnp.where(ltA, T0k, rS1k)
    L0v = jnp.where(ltA, T0v, rS1v)
    ltB = _lexless(T1k, T1v, rS0k, rS0v)
    L1k = jnp.where(ltB, T1k, rS0k)
    L1v = jnp.where(ltB, T1v, rS0v)
    lt2 = _lexless(L0k, L0v, L1k, L1v)
    P0k = jnp.where(lt2, L0k, L1k)
    P0v = jnp.where(lt2, L0v, L1v)
    P1k = jnp.where(lt2, L1k, L0k)
    P1v = jnp.where(lt2, L1v, L0v)
    T0k, T0v = plsc.sort_key_val(P0k, P0v)
    T1k, T1v = plsc.sort_key_val(P1k, P1v)
    return T0k, T0v, T1k, T1v


def _sqrt16(x):
    i = plsc.bitcast(x, jnp.int32)
    i = jnp.int32(0x5F3759DF) - (i >> 1)
    y = plsc.bitcast(i, jnp.float32)
    half_x = jnp.float32(0.5) * x
    for _ in range(3):
        y = y * (jnp.float32(1.5) - half_x * y * y)
    return x * y


def _tec_body(x_hbm, y_hbm, z_hbm, b_hbm, src_hbm, tgt_hbm, w_hbm,
              xv, yv, zv, bv, sqv, lo_t, hi_t, src_v, tgt_v, w_v):
    wid = lax.axis_index("s") * NC + lax.axis_index("c")
    r0 = wid * RPW
    iota = lax.iota(jnp.int32, L)

    pltpu.sync_copy(x_hbm, xv)
    pltpu.sync_copy(y_hbm, yv)
    pltpu.sync_copy(z_hbm, zv)
    bv[pl.ds(0, L)] = jnp.full((L,), -1, jnp.int32)
    bv[pl.ds(L + N, L)] = jnp.full((L,), NB, jnp.int32)
    pltpu.sync_copy(b_hbm, bv.at[pl.ds(L, N)])

    def sq_body(i, carry):
        off = i * L
        x = xv[pl.ds(off, L)]
        y = yv[pl.ds(off, L)]
        z = zv[pl.ds(off, L)]
        sqv[pl.ds(off, L)] = x * x + y * y + z * z
        return carry

    lax.fori_loop(0, NCHUNK, sq_body, 0)

    def bnd_body(c, carry):
        off = c * L
        cur = bv[pl.ds(L + off, L)]
        prv = bv[pl.ds(L + off - 1, L)]
        nxt = bv[pl.ds(L + off + 1, L)]
        gidx = iota + off
        plsc.store_scatter(lo_t, [cur], gidx, mask=cur != prv)
        plsc.store_scatter(hi_t, [cur], gidx + 1, mask=cur != nxt)
        return carry

    lax.fori_loop(0, NCHUNK, bnd_body, 0)

    z16 = jnp.zeros((L,), jnp.int32)
    inf16 = jnp.full((L,), INF, jnp.float32)

    def rowpair_body(rp):
        gs = [r0 + 4 * rp + j for j in range(4)]

        def row_ctx(gi):
            gisp = jnp.broadcast_to(gi, (L,))
            xi = plsc.load_gather(xv, [gisp])
            yi = plsc.load_gather(yv, [gisp])
            zi = plsc.load_gather(zv, [gisp])
            sqi = plsc.load_gather(sqv, [gisp])
            bsp = plsc.load_gather(bv, [gisp + L])
            lo = jnp.max(plsc.load_gather(lo_t, [bsp]))
            hi = jnp.max(plsc.load_gather(hi_t, [bsp]))
            return gisp, xi, yi, zi, sqi, bsp, lo, hi

        ctxs = [row_ctx(g) for g in gs]
        lo = ctxs[0][6]
        hi = ctxs[0][7]
        for ctx in ctxs[1:]:
            lo = jnp.minimum(lo, ctx[6])
            hi = jnp.maximum(hi, ctx[7])
        c0 = lo // L
        c1 = (hi + (L - 1)) // L

        def chunk16(ctx, off, vidx, bx, by, bz, bb, bsq, cvalid):
            gisp, xi, yi, zi, sqi, bsp = ctx[:6]
            dot = xi * bx + yi * by + zi * bz
            d2 = jnp.maximum((sqi + bsq) - 2.0 * dot, 0.0)
            m = (bb == bsp) & (vidx != gisp) & (d2 <= R2) & cvalid
            key = jnp.where(m, d2, INF)
            return plsc.sort_key_val(key, vidx)

        def pair_body(p, T):
            ca = c0 + 2 * p
            cb = jnp.minimum(ca + 1, NCHUNK - 1)
            bvalid = ca + 1 < c1
            offa = ca * L
            offb = cb * L
            la = (xv[pl.ds(offa, L)], yv[pl.ds(offa, L)], zv[pl.ds(offa, L)],
                  bv[pl.ds(L + offa, L)], sqv[pl.ds(offa, L)])
            lb = (xv[pl.ds(offb, L)], yv[pl.ds(offb, L)], zv[pl.ds(offb, L)],
                  bv[pl.ds(L + offb, L)], sqv[pl.ds(offb, L)])
            via = iota + offa
            vib = iota + offb
            Tn = []
            for j, ctx in enumerate(ctxs):
                Ck, Cv = chunk16(ctx, offa, via, *la, True)
                Dk, Dv = chunk16(ctx, offb, vib, *lb, bvalid)
                S = _merge16to32(Ck, Cv, Dk, Dv)
                Tn.extend(_merge32keep32(*T[4 * j:4 * j + 4], *S))
            return tuple(Tn)

        npairs = (c1 - c0 + 1) // 2
        init = (inf16, z16, inf16, z16)
        T = lax.fori_loop(0, npairs, pair_body, init * 4)

        for j, ctx in enumerate(ctxs):
            gisp_r = ctx[0]
            rr = 4 * rp + j
            T0k, T0v, T1k, T1v = T[4 * j:4 * j + 4]
            base = rr * K
            for half, (tk, tv) in enumerate(((T0k, T0v), (T1k, T1v))):
                keep = tk <= R2
                good = keep & (tk > 0.0)
                safe = jnp.where(good, tk, jnp.float32(1.0))
                wgt = jnp.where(good, _sqrt16(safe), jnp.float32(0.0))
                off = base + half * L
                src_v[pl.ds(off, L)] = jnp.where(keep, tv, -1)
                tgt_v[pl.ds(off, L)] = jnp.where(keep, gisp_r, -1)
                w_v[pl.ds(off, L)] = wgt

    plsc.parallel_loop(0, RPW // 4, 1, unroll=1)(rowpair_body)

    out0 = r0 * K
    pltpu.sync_copy(src_v, src_hbm.at[pl.ds(out0, RPW * K)])
    pltpu.sync_copy(tgt_v, tgt_hbm.at[pl.ds(out0, RPW * K)])
    pltpu.sync_copy(w_v, w_hbm.at[pl.ds(out0, RPW * K)])


@jax.jit
def kernel(pos, batch):
    x = pos[:, 0]
    y = pos[:, 1]
    z = pos[:, 2]
    b = batch.astype(jnp.int32)

    mesh = plsc.VectorSubcoreMesh(core_axis_name="c", subcore_axis_name="s",
                                  num_cores=NC)
    run = functools.partial(
        pl.kernel,
        out_type=[
            jax.ShapeDtypeStruct((N * K,), jnp.int32),
            jax.ShapeDtypeStruct((N * K,), jnp.int32),
            jax.ShapeDtypeStruct((N * K,), jnp.float32),
        ],
        mesh=mesh,
        compiler_params=pltpu.CompilerParams(needs_layout_passes=False),
        scratch_types=[
            pltpu.VMEM((N,), jnp.float32),
            pltpu.VMEM((N,), jnp.float32),
            pltpu.VMEM((N,), jnp.float32),
            pltpu.VMEM((N + 2 * L,), jnp.int32),
            pltpu.VMEM((N,), jnp.float32),
            pltpu.VMEM((NB,), jnp.int32),
            pltpu.VMEM((NB,), jnp.int32),
            pltpu.VMEM((RPW * K,), jnp.int32),
            pltpu.VMEM((RPW * K,), jnp.int32),
            pltpu.VMEM((RPW * K,), jnp.float32),
        ],
    )(_tec_body)
    src, tgt, w = run(x, y, z, b)
    edge_index = jnp.stack([src, tgt])
    return edge_index, w


# speedup vs baseline: 1.3113x; 1.3113x over previous
import functools

import jax
import jax.numpy as jnp
from jax import lax
from jax.experimental import pallas as pl
from jax.experimental.pallas import tpu as pltpu
from jax.experimental.pallas import tpu_sc as plsc

N = 4096
K = 32
R2 = 25.0
NB = 32
L = 16
NC, NS = 2, 16
NW = NC * NS
RPW = N // NW
FUSE = 4
NCHUNK = N // L
NK = N * K
INF = float("inf")


def _merge16to32_desc(Ck, Cv, Dk, Dv):
    lt = Ck < Dk
    P0k = jnp.where(lt, Ck, Dk)
    P0v = jnp.where(lt, Cv, Dv)
    P1k = jnp.where(lt, Dk, Ck)
    P1v = jnp.where(lt, Dv, Cv)
    S0k, S0v = plsc.sort_key_val(P0k, P0v, descending=True)
    S1k, S1v = plsc.sort_key_val(P1k, P1v, descending=True)
    return S0k, S0v, S1k, S1v


def _merge32keep32(T0k, T0v, T1k, T1v, S0k, S0v, S1k, S1v):
    ltA = T0k < S1k
    L0k = jnp.where(ltA, T0k, S1k)
    L0v = jnp.where(ltA, T0v, S1v)
    ltB = T1k < S0k
    L1k = jnp.where(ltB, T1k, S0k)
    L1v = jnp.where(ltB, T1v, S0v)
    lt2 = L0k < L1k
    P0k = jnp.where(lt2, L0k, L1k)
    P0v = jnp.where(lt2, L0v, L1v)
    P1k = jnp.where(lt2, L1k, L0k)
    P1v = jnp.where(lt2, L1v, L0v)
    T0k, T0v = plsc.sort_key_val(P0k, P0v)
    T1k, T1v = plsc.sort_key_val(P1k, P1v)
    return T0k, T0v, T1k, T1v


def _sqrt16(x):
    i = plsc.bitcast(x, jnp.int32)
    i = jnp.int32(0x5F3759DF) - (i >> 1)
    y = plsc.bitcast(i, jnp.float32)
    half_x = jnp.float32(0.5) * x
    for _ in range(2):
        y = y * (jnp.float32(1.5) - half_x * y * y)
    return x * y


def _tec_body(x_hbm, y_hbm, z_hbm, b_hbm, ei_hbm, w_hbm,
              xv, yv, zv, bv, sqv, lo_t, hi_t, src_v, tgt_v, w_v):
    wid = lax.axis_index("s") * NC + lax.axis_index("c")
    r0 = wid * RPW
    iota = lax.iota(jnp.int32, L)

    pltpu.sync_copy(x_hbm, xv)
    pltpu.sync_copy(y_hbm, yv)
    pltpu.sync_copy(z_hbm, zv)
    bv[pl.ds(0, L)] = jnp.full((L,), -1, jnp.int32)
    bv[pl.ds(L + N, L)] = jnp.full((L,), NB, jnp.int32)
    pltpu.sync_copy(b_hbm, bv.at[pl.ds(L, N)])

    def setup_body(c, carry):
        off = c * L
        x = xv[pl.ds(off, L)]
        y = yv[pl.ds(off, L)]
        z = zv[pl.ds(off, L)]
        sqv[pl.ds(off, L)] = x * x + y * y + z * z
        cur = bv[pl.ds(L + off, L)]
        prv = bv[pl.ds(L + off - 1, L)]
        nxt = bv[pl.ds(L + off + 1, L)]
        gidx = iota + off
        plsc.store_scatter(lo_t, [cur], gidx, mask=cur != prv)
        plsc.store_scatter(hi_t, [cur], gidx + 1, mask=cur != nxt)
        return carry

    lax.fori_loop(0, NCHUNK, setup_body, 0)

    z16 = jnp.zeros((L,), jnp.int32)
    inf16 = jnp.full((L,), INF, jnp.float32)

    def rowgroup_body(rp):
        gs = [r0 + FUSE * rp + j for j in range(FUSE)]

        def row_ctx(gi):
            gisp = jnp.broadcast_to(gi, (L,))
            xi = plsc.load_gather(xv, [gisp])
            yi = plsc.load_gather(yv, [gisp])
            zi = plsc.load_gather(zv, [gisp])
            sqi = plsc.load_gather(sqv, [gisp])
            bsp = plsc.load_gather(bv, [gisp + L])
            lo = jnp.max(plsc.load_gather(lo_t, [bsp]))
            hi = jnp.max(plsc.load_gather(hi_t, [bsp]))
            return gisp, xi, yi, zi, sqi, bsp, lo, hi

        ctxs = [row_ctx(g) for g in gs]
        lo = ctxs[0][6]
        hi = ctxs[0][7]
        for ctx in ctxs[1:]:
            lo = jnp.minimum(lo, ctx[6])
            hi = jnp.maximum(hi, ctx[7])
        c0 = lo // L
        c1 = (hi + (L - 1)) // L

        def chunk16(ctx, vidx, bx, by, bz, bb, bsq, cvalid, descending):
            gisp, xi, yi, zi, sqi, bsp = ctx[:6]
            dot = xi * bx + yi * by + zi * bz
            d2 = jnp.maximum((sqi + bsq) - 2.0 * dot, 0.0)
            m = (bb == bsp) & (vidx != gisp) & cvalid
            key = jnp.where(m, d2, INF)
            return plsc.sort_key_val(key, vidx, descending=descending)

        def pair_body(p, T):
            ca = c0 + 2 * p
            cb = jnp.minimum(ca + 1, NCHUNK - 1)
            bvalid = ca + 1 < c1
            offa = ca * L
            offb = cb * L
            la = (xv[pl.ds(offa, L)], yv[pl.ds(offa, L)], zv[pl.ds(offa, L)],
                  bv[pl.ds(L + offa, L)], sqv[pl.ds(offa, L)])
            lb = (xv[pl.ds(offb, L)], yv[pl.ds(offb, L)], zv[pl.ds(offb, L)],
                  bv[pl.ds(L + offb, L)], sqv[pl.ds(offb, L)])
            via = iota + offa
            vib = iota + offb
            Tn = []
            for j, ctx in enumerate(ctxs):
                Ck, Cv = chunk16(ctx, via, *la, True, False)
                Dk, Dv = chunk16(ctx, vib, *lb, bvalid, True)
                S = _merge16to32_desc(Ck, Cv, Dk, Dv)
                Tn.extend(_merge32keep32(*T[4 * j:4 * j + 4], *S))
            return tuple(Tn)

        npairs = (c1 - c0 + 1) // 2
        init = (inf16, z16, inf16, z16)
        T = lax.fori_loop(0, npairs, pair_body, init * FUSE)

        for j, ctx in enumerate(ctxs):
            gisp_r = ctx[0]
            rr = FUSE * rp + j
            base = rr * K
            for half in range(2):
                tk = T[4 * j + 2 * half]
                tv = T[4 * j + 2 * half + 1]
                keep = tk <= R2
                good = keep & (tk > 0.0)
                safe = jnp.where(good, tk, jnp.float32(1.0))
                wgt = jnp.where(good, _sqrt16(safe), jnp.float32(0.0))
                off = base + half * L
                src_v[pl.ds(off, L)] = jnp.where(keep, tv, -1)
                tgt_v[pl.ds(off, L)] = jnp.where(keep, gisp_r, -1)
                w_v[pl.ds(off, L)] = wgt

    plsc.parallel_loop(0, RPW // FUSE, 1, unroll=1)(rowgroup_body)

    out0 = r0 * K
    pltpu.sync_copy(src_v, ei_hbm.at[pl.ds(out0, RPW * K)])
    pltpu.sync_copy(tgt_v, ei_hbm.at[pl.ds(NK + out0, RPW * K)])
    pltpu.sync_copy(w_v, w_hbm.at[pl.ds(out0, RPW * K)])


@jax.jit
def kernel(pos, batch):
    x = pos[:, 0]
    y = pos[:, 1]
    z = pos[:, 2]
    b = batch.astype(jnp.int32)

    mesh = plsc.VectorSubcoreMesh(core_axis_name="c", subcore_axis_name="s",
                                  num_cores=NC)
    run = functools.partial(
        pl.kernel,
        out_type=[
            jax.ShapeDtypeStruct((2 * NK,), jnp.int32),
            jax.ShapeDtypeStruct((NK,), jnp.float32),
        ],
        mesh=mesh,
        compiler_params=pltpu.CompilerParams(needs_layout_passes=False),
        scratch_types=[
            pltpu.VMEM((N,), jnp.float32),
            pltpu.VMEM((N,), jnp.float32),
            pltpu.VMEM((N,), jnp.float32),
            pltpu.VMEM((N + 2 * L,), jnp.int32),
            pltpu.VMEM((N,), jnp.float32),
            pltpu.VMEM((NB,), jnp.int32),
            pltpu.VMEM((NB,), jnp.int32),
            pltpu.VMEM((RPW * K,), jnp.int32),
            pltpu.VMEM((RPW * K,), jnp.int32),
            pltpu.VMEM((RPW * K,), jnp.float32),
        ],
    )(_tec_body)
    ei_flat, w = run(x, y, z, b)
    return ei_flat.reshape(2, NK), w


# speedup vs baseline: 1.4488x; 1.1049x over previous
import functools

import jax
import jax.numpy as jnp
from jax import lax
from jax.experimental import pallas as pl
from jax.experimental.pallas import tpu as pltpu
from jax.experimental.pallas import tpu_sc as plsc

N = 4096
K = 32
R2 = 25.0
NB = 32
L = 16
NC, NS = 2, 16
NW = NC * NS
RPW = N // NW
FUSE = 4
NCHUNK = N // L
NK = N * K
INF = float("inf")


def _merge16to32_desc(Ck, Cv, Dk, Dv):
    lt = Ck < Dk
    P0k = jnp.where(lt, Ck, Dk)
    P0v = jnp.where(lt, Cv, Dv)
    P1k = jnp.where(lt, Dk, Ck)
    P1v = jnp.where(lt, Dv, Cv)
    S0k, S0v = plsc.sort_key_val(P0k, P0v, descending=True)
    S1k, S1v = plsc.sort_key_val(P1k, P1v, descending=True)
    return S0k, S0v, S1k, S1v


def _merge32keep32(T0k, T0v, T1k, T1v, S0k, S0v, S1k, S1v):
    ltA = T0k < S1k
    L0k = jnp.where(ltA, T0k, S1k)
    L0v = jnp.where(ltA, T0v, S1v)
    ltB = T1k < S0k
    L1k = jnp.where(ltB, T1k, S0k)
    L1v = jnp.where(ltB, T1v, S0v)
    lt2 = L0k < L1k
    P0k = jnp.where(lt2, L0k, L1k)
    P0v = jnp.where(lt2, L0v, L1v)
    P1k = jnp.where(lt2, L1k, L0k)
    P1v = jnp.where(lt2, L1v, L0v)
    T0k, T0v = plsc.sort_key_val(P0k, P0v)
    T1k, T1v = plsc.sort_key_val(P1k, P1v)
    return T0k, T0v, T1k, T1v


def _sqrt16(x):
    i = plsc.bitcast(x, jnp.int32)
    i = jnp.int32(0x5F3759DF) - (i >> 1)
    y = plsc.bitcast(i, jnp.float32)
    half_x = jnp.float32(0.5) * x
    for _ in range(2):
        y = y * (jnp.float32(1.5) - half_x * y * y)
    return x * y


def _tec_body(x_hbm, y_hbm, z_hbm, b_hbm, ei_hbm, w_hbm,
              xv, yv, zv, bv, sqv, lo_t, hi_t, src_v, tgt_v, w_v):
    wid = lax.axis_index("s") * NC + lax.axis_index("c")
    r0 = wid * RPW
    iota = lax.iota(jnp.int32, L)

    pltpu.sync_copy(x_hbm, xv)
    pltpu.sync_copy(y_hbm, yv)
    pltpu.sync_copy(z_hbm, zv)
    bv[pl.ds(0, L)] = jnp.full((L,), -1, jnp.int32)
    bv[pl.ds(L + N, L)] = jnp.full((L,), NB, jnp.int32)
    pltpu.sync_copy(b_hbm, bv.at[pl.ds(L, N)])

    def setup_body(c, carry):
        off = c * L
        x = xv[pl.ds(off, L)]
        y = yv[pl.ds(off, L)]
        z = zv[pl.ds(off, L)]
        sqv[pl.ds(off, L)] = x * x + y * y + z * z
        xv[pl.ds(off, L)] = -2.0 * x
        yv[pl.ds(off, L)] = -2.0 * y
        zv[pl.ds(off, L)] = -2.0 * z
        cur = bv[pl.ds(L + off, L)]
        prv = bv[pl.ds(L + off - 1, L)]
        nxt = bv[pl.ds(L + off + 1, L)]
        gidx = iota + off
        plsc.store_scatter(lo_t, [cur], gidx, mask=cur != prv)
        plsc.store_scatter(hi_t, [cur], gidx + 1, mask=cur != nxt)
        return carry

    lax.fori_loop(0, NCHUNK, setup_body, 0)

    z16 = jnp.zeros((L,), jnp.int32)
    inf16 = jnp.full((L,), INF, jnp.float32)

    def rowgroup_body(rp):
        gs = [r0 + FUSE * rp + j for j in range(FUSE)]

        def row_ctx(gi):
            gisp = jnp.broadcast_to(gi, (L,))
            xi = jnp.float32(-0.5) * plsc.load_gather(xv, [gisp])
            yi = jnp.float32(-0.5) * plsc.load_gather(yv, [gisp])
            zi = jnp.float32(-0.5) * plsc.load_gather(zv, [gisp])
            sqi = plsc.load_gather(sqv, [gisp])
            bsp = plsc.load_gather(bv, [gisp + L])
            lo = jnp.max(plsc.load_gather(lo_t, [bsp]))
            hi = jnp.max(plsc.load_gather(hi_t, [bsp]))
            return gisp, xi, yi, zi, sqi, bsp, lo, hi

        ctxs = [row_ctx(g) for g in gs]
        lo = ctxs[0][6]
        hi = ctxs[0][7]
        for ctx in ctxs[1:]:
            lo = jnp.minimum(lo, ctx[6])
            hi = jnp.maximum(hi, ctx[7])
        c0 = lo // L
        c1 = (hi + (L - 1)) // L

        def chunk16(ctx, vidx, bx, by, bz, bb, bsq, cvalid, descending):
            gisp, xi, yi, zi, sqi, bsp = ctx[:6]
            dot = xi * bx + yi * by + zi * bz
            d2 = jnp.maximum((sqi + bsq) + dot, 0.0)
            m = (bb == bsp) & (vidx != gisp) & cvalid
            key = jnp.where(m, d2, INF)
            return plsc.sort_key_val(key, vidx, descending=descending)

        def pair_body(p, T):
            ca = c0 + 2 * p
            cb = jnp.minimum(ca + 1, NCHUNK - 1)
            bvalid = ca + 1 < c1
            offa = ca * L
            offb = cb * L
            la = (xv[pl.ds(offa, L)], yv[pl.ds(offa, L)], zv[pl.ds(offa, L)],
                  bv[pl.ds(L + offa, L)], sqv[pl.ds(offa, L)])
            lb = (xv[pl.ds(offb, L)], yv[pl.ds(offb, L)], zv[pl.ds(offb, L)],
                  bv[pl.ds(L + offb, L)], sqv[pl.ds(offb, L)])
            via = iota + offa
            vib = iota + offb
            Tn = []
            for j, ctx in enumerate(ctxs):
                Ck, Cv = chunk16(ctx, via, *la, True, False)
                Dk, Dv = chunk16(ctx, vib, *lb, bvalid, True)
                S = _merge16to32_desc(Ck, Cv, Dk, Dv)
                Tn.extend(_merge32keep32(*T[4 * j:4 * j + 4], *S))
            return tuple(Tn)

        npairs = (c1 - c0 + 1) // 2
        init = (inf16, z16, inf16, z16)
        T = lax.fori_loop(0, npairs, pair_body, init * FUSE)

        for j, ctx in enumerate(ctxs):
            gisp_r = ctx[0]
            rr = FUSE * rp + j
            base = rr * K
            for half in range(2):
                tk = T[4 * j + 2 * half]
                tv = T[4 * j + 2 * half + 1]
                keep = tk <= R2
                good = keep & (tk > 0.0)
                safe = jnp.where(good, tk, jnp.float32(1.0))
                wgt = jnp.where(good, _sqrt16(safe), jnp.float32(0.0))
                off = base + half * L
                src_v[pl.ds(off, L)] = jnp.where(keep, tv, -1)
                tgt_v[pl.ds(off, L)] = jnp.where(keep, gisp_r, -1)
                w_v[pl.ds(off, L)] = wgt

    plsc.parallel_loop(0, RPW // FUSE, 1, unroll=1)(rowgroup_body)

    out0 = r0 * K
    pltpu.sync_copy(src_v, ei_hbm.at[0, pl.ds(out0, RPW * K)])
    pltpu.sync_copy(tgt_v, ei_hbm.at[1, pl.ds(out0, RPW * K)])
    pltpu.sync_copy(w_v, w_hbm.at[pl.ds(out0, RPW * K)])


@jax.jit
def kernel(pos, batch):
    x = pos[:, 0]
    y = pos[:, 1]
    z = pos[:, 2]
    b = batch.astype(jnp.int32)

    mesh = plsc.VectorSubcoreMesh(core_axis_name="c", subcore_axis_name="s",
                                  num_cores=NC)
    run = functools.partial(
        pl.kernel,
        out_type=[
            jax.ShapeDtypeStruct((2, NK), jnp.int32),
            jax.ShapeDtypeStruct((NK,), jnp.float32),
        ],
        mesh=mesh,
        compiler_params=pltpu.CompilerParams(needs_layout_passes=False),
        scratch_types=[
            pltpu.VMEM((N,), jnp.float32),
            pltpu.VMEM((N,), jnp.float32),
            pltpu.VMEM((N,), jnp.float32),
            pltpu.VMEM((N + 2 * L,), jnp.int32),
            pltpu.VMEM((N,), jnp.float32),
            pltpu.VMEM((NB,), jnp.int32),
            pltpu.VMEM((NB,), jnp.int32),
            pltpu.VMEM((RPW * K,), jnp.int32),
            pltpu.VMEM((RPW * K,), jnp.int32),
            pltpu.VMEM((RPW * K,), jnp.float32),
        ],
    )(_tec_body)
    ei, w = run(x, y, z, b)
    return ei, w


# speedup vs baseline: 1.5174x; 1.0474x over previous
import functools

import jax
import jax.numpy as jnp
from jax import lax
from jax.experimental import pallas as pl
from jax.experimental.pallas import tpu as pltpu
from jax.experimental.pallas import tpu_sc as plsc

N = 4096
K = 32
R2 = 25.0
NB = 32
L = 16
NC, NS = 2, 16
NW = NC * NS
RPW = N // NW
FUSE = 4
NCHUNK = N // L
NK = N * K
INF = float("inf")


def _merge16to32_desc(Ck, Cv, Dk, Dv):
    lt = Ck < Dk
    P0k = jnp.where(lt, Ck, Dk)
    P0v = jnp.where(lt, Cv, Dv)
    P1k = jnp.where(lt, Dk, Ck)
    P1v = jnp.where(lt, Dv, Cv)
    S0k, S0v = plsc.sort_key_val(P0k, P0v, descending=True)
    S1k, S1v = plsc.sort_key_val(P1k, P1v, descending=True)
    return S0k, S0v, S1k, S1v


def _merge32keep32(T0k, T0v, T1k, T1v, S0k, S0v, S1k, S1v):
    ltA = T0k < S1k
    L0k = jnp.where(ltA, T0k, S1k)
    L0v = jnp.where(ltA, T0v, S1v)
    ltB = T1k < S0k
    L1k = jnp.where(ltB, T1k, S0k)
    L1v = jnp.where(ltB, T1v, S0v)
    lt2 = L0k < L1k
    P0k = jnp.where(lt2, L0k, L1k)
    P0v = jnp.where(lt2, L0v, L1v)
    P1k = jnp.where(lt2, L1k, L0k)
    P1v = jnp.where(lt2, L1v, L0v)
    T0k, T0v = plsc.sort_key_val(P0k, P0v)
    T1k, T1v = plsc.sort_key_val(P1k, P1v)
    return T0k, T0v, T1k, T1v


def _sqrt16(x):
    i = plsc.bitcast(x, jnp.int32)
    i = jnp.int32(0x5F3759DF) - (i >> 1)
    y = plsc.bitcast(i, jnp.float32)
    half_x = jnp.float32(0.5) * x
    for _ in range(2):
        y = y * (jnp.float32(1.5) - half_x * y * y)
    return x * y


def _tec_body(x_hbm, y_hbm, z_hbm, b_hbm, ei_hbm, w_hbm,
              xv, yv, zv, bv, sqv, lo_t, hi_t, src_v, tgt_v, w_v, dma_sem):
    wid = lax.axis_index("s") * NC + lax.axis_index("c")
    r0 = wid * RPW
    iota = lax.iota(jnp.int32, L)

    copies = [
        pltpu.async_copy(x_hbm, xv, dma_sem),
        pltpu.async_copy(y_hbm, yv, dma_sem),
        pltpu.async_copy(z_hbm, zv, dma_sem),
        pltpu.async_copy(b_hbm, bv.at[pl.ds(L, N)], dma_sem),
    ]
    bv[pl.ds(0, L)] = jnp.full((L,), -1, jnp.int32)
    bv[pl.ds(L + N, L)] = jnp.full((L,), NB, jnp.int32)
    for c in copies:
        c.wait()

    def setup_body(c, carry):
        off = c * L
        x = xv[pl.ds(off, L)]
        y = yv[pl.ds(off, L)]
        z = zv[pl.ds(off, L)]
        sqv[pl.ds(off, L)] = x * x + y * y + z * z
        xv[pl.ds(off, L)] = -2.0 * x
        yv[pl.ds(off, L)] = -2.0 * y
        zv[pl.ds(off, L)] = -2.0 * z
        cur = bv[pl.ds(L + off, L)]
        prv = bv[pl.ds(L + off - 1, L)]
        nxt = bv[pl.ds(L + off + 1, L)]
        gidx = iota + off
        plsc.store_scatter(lo_t, [cur], gidx, mask=cur != prv)
        plsc.store_scatter(hi_t, [cur], gidx + 1, mask=cur != nxt)
        return carry

    lax.fori_loop(0, NCHUNK, setup_body, 0)

    z16 = jnp.zeros((L,), jnp.int32)
    inf16 = jnp.full((L,), INF, jnp.float32)

    def rowgroup_body(rp):
        gs = [r0 + FUSE * rp + j for j in range(FUSE)]

        def row_ctx(gi):
            gisp = jnp.broadcast_to(gi, (L,))
            xi = jnp.float32(-0.5) * plsc.load_gather(xv, [gisp])
            yi = jnp.float32(-0.5) * plsc.load_gather(yv, [gisp])
            zi = jnp.float32(-0.5) * plsc.load_gather(zv, [gisp])
            sqi = plsc.load_gather(sqv, [gisp])
            bsp = plsc.load_gather(bv, [gisp + L])
            return gisp, xi, yi, zi, sqi, bsp

        ctxs = [row_ctx(g) for g in gs]
        lo = jnp.max(plsc.load_gather(lo_t, [ctxs[0][5]]))
        hi = jnp.max(plsc.load_gather(hi_t, [ctxs[-1][5]]))
        c0 = lo // L
        c1 = (hi + (L - 1)) // L

        def chunk16(ctx, vidx, bx, by, bz, bb, bsq, cvalid, descending):
            gisp, xi, yi, zi, sqi, bsp = ctx[:6]
            dot = xi * bx + yi * by + zi * bz
            d2 = jnp.maximum((sqi + bsq) + dot, 0.0)
            m = (bb == bsp) & (vidx != gisp) & cvalid
            key = jnp.where(m, d2, INF)
            return plsc.sort_key_val(key, vidx, descending=descending)

        def pair_body(p, T):
            ca = c0 + 2 * p
            cb = jnp.minimum(ca + 1, NCHUNK - 1)
            bvalid = ca + 1 < c1
            offa = ca * L
            offb = cb * L
            la = (xv[pl.ds(offa, L)], yv[pl.ds(offa, L)], zv[pl.ds(offa, L)],
                  bv[pl.ds(L + offa, L)], sqv[pl.ds(offa, L)])
            lb = (xv[pl.ds(offb, L)], yv[pl.ds(offb, L)], zv[pl.ds(offb, L)],
                  bv[pl.ds(L + offb, L)], sqv[pl.ds(offb, L)])
            via = iota + offa
            vib = iota + offb
            Tn = []
            for j, ctx in enumerate(ctxs):
                Ck, Cv = chunk16(ctx, via, *la, True, False)
                Dk, Dv = chunk16(ctx, vib, *lb, bvalid, True)
                S = _merge16to32_desc(Ck, Cv, Dk, Dv)
                Tn.extend(_merge32keep32(*T[4 * j:4 * j + 4], *S))
            return tuple(Tn)

        npairs = (c1 - c0 + 1) // 2
        init = (inf16, z16, inf16, z16)
        T = lax.fori_loop(0, npairs, pair_body, init * FUSE)

        for j, ctx in enumerate(ctxs):
            gisp_r = ctx[0]
            rr = FUSE * rp + j
            base = rr * K
            for half in range(2):
                tk = T[4 * j + 2 * half]
                tv = T[4 * j + 2 * half + 1]
                keep = tk <= R2
                good = keep & (tk > 0.0)
                safe = jnp.where(good, tk, jnp.float32(1.0))
                wgt = jnp.where(good, _sqrt16(safe), jnp.float32(0.0))
                off = base + half * L
                src_v[pl.ds(off, L)] = jnp.where(keep, tv, -1)
                tgt_v[pl.ds(off, L)] = jnp.where(keep, gisp_r, -1)
                w_v[pl.ds(off, L)] = wgt

    plsc.parallel_loop(0, RPW // FUSE, 1, unroll=1)(rowgroup_body)

    out0 = r0 * K
    pltpu.sync_copy(src_v, ei_hbm.at[0, pl.ds(out0, RPW * K)])
    pltpu.sync_copy(tgt_v, ei_hbm.at[1, pl.ds(out0, RPW * K)])
    pltpu.sync_copy(w_v, w_hbm.at[pl.ds(out0, RPW * K)])


@jax.jit
def kernel(pos, batch):
    x = pos[:, 0]
    y = pos[:, 1]
    z = pos[:, 2]
    b = batch.astype(jnp.int32)

    mesh = plsc.VectorSubcoreMesh(core_axis_name="c", subcore_axis_name="s",
                                  num_cores=NC)
    run = functools.partial(
        pl.kernel,
        out_type=[
            jax.ShapeDtypeStruct((2, NK), jnp.int32),
            jax.ShapeDtypeStruct((NK,), jnp.float32),
        ],
        mesh=mesh,
        compiler_params=pltpu.CompilerParams(needs_layout_passes=False),
        scratch_types=[
            pltpu.VMEM((N,), jnp.float32),
            pltpu.VMEM((N,), jnp.float32),
            pltpu.VMEM((N,), jnp.float32),
            pltpu.VMEM((N + 2 * L,), jnp.int32),
            pltpu.VMEM((N,), jnp.float32),
            pltpu.VMEM((NB,), jnp.int32),
            pltpu.VMEM((NB,), jnp.int32),
            pltpu.VMEM((RPW * K,), jnp.int32),
            pltpu.VMEM((RPW * K,), jnp.int32),
            pltpu.VMEM((RPW * K,), jnp.float32),
            pltpu.SemaphoreType.DMA,
        ],
    )(_tec_body)
    ei, w = run(x, y, z, b)
    return ei, w
